# Initial kernel scaffold; baseline (speedup 1.0000x reference)
import jax
import jax.numpy as jnp
from jax.experimental import pallas as pl


def kernel(x, edge_index, W1, b1, W2, b2, W3, b3):
    raise NotImplementedError("write your pallas kernel here")


# speedup vs baseline: 5.3535x; 5.3535x over previous
import functools

import jax
import jax.numpy as jnp
from jax import lax
from jax.experimental import pallas as pl
from jax.experimental.pallas import tpu as pltpu
from jax.experimental.pallas import tpu_sc as plsc

N = 10000
E = 320000
D = 128
NC = 2
NS = 16
NW = NC * NS
EPW = E // NW
CH = 80
NCHUNK = EPW // CH
RPT = 640
NA = NS * RPT
ZR = 160
CR = NA // D

_SC_MESH = plsc.VectorSubcoreMesh(core_axis_name="c", subcore_axis_name="s")


def _make_sc_spmm(with_counts):
    feat_t = jax.ShapeDtypeStruct((NC, NA, D), jnp.float32)
    out_type = (feat_t, jax.ShapeDtypeStruct((NC, CR, D), jnp.float32)) \
        if with_counts else feat_t
    scratch = [
        pltpu.VMEM_SHARED((NA, D), jnp.float32),
        pltpu.VMEM((ZR, D), jnp.float32),
        pltpu.VMEM((CH,), jnp.int32),
        pltpu.VMEM((CH,), jnp.int32),
        pltpu.VMEM((CH, D), jnp.float32),
        pltpu.SemaphoreType.DMA,
    ]
    if with_counts:
        scratch += [
            pltpu.VMEM_SHARED((CR, D), jnp.float32),
            pltpu.VMEM((CR, D), jnp.float32),
            pltpu.VMEM((CR,), jnp.int32),
        ]

    def body(x_hbm, src_hbm, dst_hbm, zeros_hbm, *rest):
        if with_counts:
            (iota_hbm, feat_out, cnt_out,
             acc, zbuf, sbuf, dbuf, rows, gsem, acc_cnt, cnt, iota_v) = rest
        else:
            feat_out, acc, zbuf, sbuf, dbuf, rows, gsem = rest
        c = lax.axis_index("c")
        s = lax.axis_index("s")
        wid = c * NS + s

        pltpu.sync_copy(zeros_hbm, zbuf)
        for j in range(RPT // ZR):
            pltpu.sync_copy(zbuf, acc.at[pl.ds(s * RPT + j * ZR, ZR)])
        if with_counts:
            z16 = jnp.zeros((16,), jnp.float32)

            def zrow(r, _):
                for j in range(D // 16):
                    cnt[r, pl.ds(j * 16, 16)] = z16
                return _

            lax.fori_loop(0, CR, zrow, None)
            pltpu.sync_copy(iota_hbm, iota_v)
            @pl.when(s < CR // 8)
            def _():
                pltpu.sync_copy(zbuf.at[pl.ds(0, 8)],
                                acc_cnt.at[pl.ds(s * 8, 8)])
        plsc.subcore_barrier()

        ones16 = jnp.full((16,), 1.0, jnp.float32)

        def chunk(k, _):
            off = pl.multiple_of(wid * EPW + k * CH, CH)
            pltpu.sync_copy(src_hbm.at[pl.ds(off, CH)], sbuf)
            pltpu.sync_copy(dst_hbm.at[pl.ds(off, CH)], dbuf)
            pltpu.async_copy(x_hbm.at[sbuf], rows, gsem).wait()
            pltpu.sync_copy(rows, acc.at[dbuf], add=True)
            if with_counts:
                for g in range(CH // 16):
                    idx = dbuf[pl.ds(g * 16, 16)]
                    plsc.addupdate_scatter(
                        cnt, [lax.shift_right_logical(idx, 7),
                              lax.bitwise_and(idx, 127)], ones16)
            return _

        lax.fori_loop(0, NCHUNK, chunk, None)

        if with_counts:
            pltpu.sync_copy(cnt, acc_cnt.at[iota_v], add=True)
        plsc.subcore_barrier()

        pltpu.sync_copy(acc.at[pl.ds(s * RPT, RPT)],
                        feat_out.at[c, pl.ds(s * RPT, RPT)])
        if with_counts:
            @pl.when(s < CR // 8)
            def _():
                pltpu.sync_copy(acc_cnt.at[pl.ds(s * 8, 8)],
                                cnt_out.at[c, pl.ds(s * 8, 8)])

    return pl.kernel(
        body, out_type=out_type, mesh=_SC_MESH, scratch_types=scratch,
        compiler_params=pltpu.CompilerParams(needs_layout_passes=False))


_sc_spmm_counts = _make_sc_spmm(True)
_sc_spmm = _make_sc_spmm(False)

_BR = 1000


def _tc_layer_body(S_ref, cnt_ref, W_ref, b_ref, out_ref):
    s = S_ref[0] + S_ref[1]
    cnt = cnt_ref[...]
    denom = jnp.maximum(cnt, 1.0)
    mask = (cnt > 0.0).astype(jnp.float32)
    h = lax.dot_general(s, W_ref[...], (((1,), (1,)), ((), ())),
                        preferred_element_type=jnp.float32)
    out_ref[...] = jnp.maximum(h / denom + mask * b_ref[...], 0.0)


def _tc_layer(S, cnt, W, b):
    return pl.pallas_call(
        _tc_layer_body,
        grid=(N // _BR,),
        in_specs=[
            pl.BlockSpec((NC, _BR, D), lambda i: (0, i, 0)),
            pl.BlockSpec((_BR, 1), lambda i: (i, 0)),
            pl.BlockSpec((D, D), lambda i: (0, 0)),
            pl.BlockSpec((1, D), lambda i: (0, 0)),
        ],
        out_specs=pl.BlockSpec((_BR, D), lambda i: (i, 0)),
        out_shape=jax.ShapeDtypeStruct((N, D), jnp.float32),
    )(S, cnt, W, b.reshape(1, D))


def _tc_final_body(S_ref, cnt_ref, W2_ref, b2_ref, W3_ref, b3_ref, out_ref):
    s = S_ref[0] + S_ref[1]
    cnt = cnt_ref[...]
    denom = jnp.maximum(cnt, 1.0)
    mask = (cnt > 0.0).astype(jnp.float32)
    h = lax.dot_general(s, W2_ref[...], (((1,), (1,)), ((), ())),
                        preferred_element_type=jnp.float32)
    h = jnp.maximum(h / denom + mask * b2_ref[...], 0.0)
    out_ref[...] = lax.dot_general(h, W3_ref[...], (((1,), (1,)), ((), ())),
                                   preferred_element_type=jnp.float32) + b3_ref[...]


def _tc_final(S, cnt, W2, b2, W3, b3):
    return pl.pallas_call(
        _tc_final_body,
        grid=(N // _BR,),
        in_specs=[
            pl.BlockSpec((NC, _BR, D), lambda i: (0, i, 0)),
            pl.BlockSpec((_BR, 1), lambda i: (i, 0)),
            pl.BlockSpec((D, D), lambda i: (0, 0)),
            pl.BlockSpec((1, D), lambda i: (0, 0)),
            pl.BlockSpec((D, D), lambda i: (0, 0)),
            pl.BlockSpec((1, D), lambda i: (0, 0)),
        ],
        out_specs=pl.BlockSpec((_BR, D), lambda i: (i, 0)),
        out_shape=jax.ShapeDtypeStruct((N, D), jnp.float32),
    )(S, cnt, W2, b2.reshape(1, D), W3, b3.reshape(1, D))


def kernel(x, edge_index, W1, b1, W2, b2, W3, b3):
    src = edge_index[0].astype(jnp.int32)
    dst = edge_index[1].astype(jnp.int32)
    zeros_blk = jnp.zeros((ZR, D), jnp.float32)
    iota_cr = jnp.arange(CR, dtype=jnp.int32)

    S1, C = _sc_spmm_counts(x, src, dst, zeros_blk, iota_cr)
    cnt = (C[0] + C[1]).reshape(NA, 1)[:N]
    h1 = _tc_layer(S1, cnt, W1, b1)
    S2 = _sc_spmm(h1, src, dst, zeros_blk)
    return _tc_final(S2, cnt, W2, b2, W3, b3)
